# Initial kernel scaffold; baseline (speedup 1.0000x reference)
import jax
import jax.numpy as jnp
from jax.experimental import pallas as pl


def kernel(x, lin_weight_0, src_weight_0, dst_weight_0, bias_weight_0, lin_weight_1, src_weight_1, dst_weight_1, bias_weight_1):
    raise NotImplementedError("write your pallas kernel here")


# speedup vs baseline: 16966.0851x; 16966.0851x over previous
import jax
import jax.numpy as jnp
from jax.experimental import pallas as pl
from functools import partial

N = 2048


def _layer(h, lin_w, asrc, adst, bias):
    hh = jnp.dot(h, lin_w.T, preferred_element_type=jnp.float32)
    s_col = jnp.dot(hh, asrc, preferred_element_type=jnp.float32)
    d_col = jnp.dot(hh, adst, preferred_element_type=jnp.float32)
    m1 = jnp.max(s_col)
    e1 = jnp.exp(s_col - m1)
    e02 = jnp.exp(0.2 * (s_col - m1))
    hh0 = hh[:, 0:1]
    hh1 = hh[:, 1:2]
    V = jnp.concatenate(
        [e02, e02 * hh0, e02 * hh1, e1, e1 * hh0, e1 * hh1], axis=1)
    s_row = jnp.transpose(s_col)
    M = (s_row + d_col <= 0.0).astype(jnp.float32)
    W = jnp.dot(M, V, preferred_element_type=jnp.float32)
    tot = jnp.sum(V[:, 3:6], axis=0, keepdims=True)
    W_neg = W[:, 0:3]
    W_pos = tot - W[:, 3:6]
    b2 = d_col + m1
    b1 = 0.2 * b2
    L = jnp.maximum(b1, b2)
    f1 = jnp.exp(b1 - L)
    f2 = jnp.exp(b2 - L)
    den = f1 * W_neg[:, 0:1] + f2 * W_pos[:, 0:1]
    num0 = f1 * W_neg[:, 1:2] + f2 * W_pos[:, 1:2]
    num1 = f1 * W_neg[:, 2:3] + f2 * W_pos[:, 2:3]
    out = jnp.concatenate([num0 / den, num1 / den], axis=1) + bias
    return out


def _gnn_kernel(x_ref, lw0_ref, as0_ref, ad0_ref, b0_ref,
                lw1_ref, as1_ref, ad1_ref, b1_ref, out_ref):
    x0 = x_ref[...]
    xpos = jax.lax.broadcasted_iota(jnp.int32, (N, 1), 0).astype(jnp.float32) - N / 2
    h = jnp.concatenate([x0, xpos], axis=1)
    h = _layer(h, lw0_ref[...], as0_ref[...], ad0_ref[...], b0_ref[...])
    h = _layer(h, lw1_ref[...], as1_ref[...], ad1_ref[...], b1_ref[...])
    out_ref[...] = h


@jax.jit
def kernel(x, lin_weight_0, src_weight_0, dst_weight_0, bias_weight_0,
           lin_weight_1, src_weight_1, dst_weight_1, bias_weight_1):
    x_col = x.reshape(N, 1)
    args = (
        x_col,
        lin_weight_0, src_weight_0.reshape(2, 1), dst_weight_0.reshape(2, 1),
        bias_weight_0.reshape(1, 2),
        lin_weight_1, src_weight_1.reshape(2, 1), dst_weight_1.reshape(2, 1),
        bias_weight_1.reshape(1, 2),
    )
    return pl.pallas_call(
        _gnn_kernel,
        out_shape=jax.ShapeDtypeStruct((N, 2), jnp.float32),
    )(*args)
